# baseline (speedup 1.0000x reference)
import functools

import jax
import jax.numpy as jnp
from jax import lax
from jax.experimental import pallas as pl
from jax.experimental.pallas import tpu as pltpu
from jax.experimental.pallas import tpu_sc as plsc

_B, _N, _E, _NIN, _NHID, _NOUT = 4, 1024, 16384, 128, 128, 128
_NW = 32
_ROWS_PER_W = _N // _NW
_BLK = _ROWS_PER_W * _N
_TE = 1024
_LANES = 16


def _build_adjacency(edge_index):
    mesh = plsc.VectorSubcoreMesh(core_axis_name="c", subcore_axis_name="s")

    @functools.partial(
        pl.kernel,
        out_type=jax.ShapeDtypeStruct((_N * _N,), jnp.float32),
        mesh=mesh,
        compiler_params=pltpu.CompilerParams(needs_layout_passes=False),
        scratch_types=[
            pltpu.VMEM((_E,), jnp.int32),
            pltpu.VMEM((_E,), jnp.int32),
            pltpu.VMEM((_BLK,), jnp.float32),
        ],
    )
    def scatter_kernel(edge_hbm, a_hbm, src_v, dst_v, blk_v):
        wid = lax.axis_index("s") * 2 + lax.axis_index("c")
        lo = wid * _ROWS_PER_W
        pltpu.sync_copy(edge_hbm.at[0], src_v)
        pltpu.sync_copy(edge_hbm.at[1], dst_v)

        @plsc.parallel_loop(0, _BLK // _LANES, unroll=16)
        def zero_body(i):
            blk_v[pl.ds(i * _LANES, _LANES)] = jnp.zeros((_LANES,), jnp.float32)

        ones = jnp.ones((_LANES,), jnp.float32)

        @plsc.parallel_loop(0, _E // _LANES, unroll=8)
        def scat_body(i):
            s = src_v[pl.ds(i * _LANES, _LANES)]
            d = dst_v[pl.ds(i * _LANES, _LANES)]
            rel = d - lo
            msk = (rel >= 0) & (rel < _ROWS_PER_W)
            idx = jnp.where(msk, rel * _N + s, 0)
            plsc.addupdate_scatter(blk_v, [idx], ones, mask=msk)
        pltpu.sync_copy(blk_v, a_hbm.at[pl.ds(lo * _N, _BLK)])

    return scatter_kernel(edge_index)


_NT = _E // _TE
_GRID = _NT + 2


def _fused_body(a_ref, x_ref, min_ref, mout_ref, w1_ref, b1_ref, wm_ref,
                bm_ref, w2_ref, b2_ref, out_ref, h_ref, dinv_ref, acc_ref):
    i = pl.program_id(0)

    @pl.when(i == 0)
    def _stage1():
        a = a_ref[...]
        deg = jnp.sum(a, axis=1) + 1.0
        dinv = lax.rsqrt(deg)
        w1 = w1_ref[...]
        p = jnp.concatenate(
            [lax.dot_general(x_ref[b], w1, (((1,), (1,)), ((), ())),
                             preferred_element_type=jnp.float32)
             for b in range(_B)], axis=1)
        q = p * dinv[:, None]
        agg = jnp.dot(a, q, preferred_element_type=jnp.float32) + q
        b1t = jnp.concatenate([b1_ref[...][None, :]] * _B, axis=1)
        h = jnp.maximum(agg * dinv[:, None] + b1t, 0.0)
        h_ref[...] = h.astype(jnp.bfloat16)
        dinv_ref[...] = dinv[:, None]

    @pl.when((i >= 1) & (i <= _NT))
    def _stage2():
        m_in_t = min_ref[...].astype(jnp.bfloat16)
        m_out_t = mout_ref[...].astype(jnp.bfloat16)
        h = h_ref[...]
        recv = jnp.dot(m_in_t, h, preferred_element_type=jnp.float32)
        send = jnp.dot(m_out_t, h, preferred_element_type=jnp.float32)
        wm = wm_ref[...]
        wma = wm[:, :_NHID]
        wmb = wm[:, _NHID:]
        bm = bm_ref[...][None, :]
        e_blocks = []
        for b in range(_B):
            rb = recv[:, b * _NHID:(b + 1) * _NHID]
            sb = send[:, b * _NHID:(b + 1) * _NHID]
            eb = (lax.dot_general(rb, wma, (((1,), (1,)), ((), ())),
                                  preferred_element_type=jnp.float32)
                  + lax.dot_general(sb, wmb, (((1,), (1,)), ((), ())),
                                    preferred_element_type=jnp.float32))
            e_blocks.append(jnp.maximum(eb + bm, 0.0))
        e_t = jnp.concatenate(e_blocks, axis=1).astype(jnp.bfloat16)
        contrib = lax.dot_general(m_in_t, e_t, (((0,), (0,)), ((), ())),
                                  preferred_element_type=jnp.float32)

        @pl.when(i == 1)
        def _():
            acc_ref[...] = contrib

        @pl.when(i > 1)
        def _():
            acc_ref[...] += contrib

    @pl.when(i == _NT + 1)
    def _stage3():
        dinv = dinv_ref[...][:, 0]
        node = acc_ref[...] * (1.0 / _N)
        w2 = w2_ref[...]
        q = jnp.concatenate(
            [lax.dot_general(node[:, b * _NHID:(b + 1) * _NHID], w2,
                             (((1,), (1,)), ((), ())),
                             preferred_element_type=jnp.float32)
             for b in range(_B)], axis=1)
        qq = q * dinv[:, None]
        r = jnp.dot(a_ref[...], qq, preferred_element_type=jnp.float32) + qq
        r = r * dinv[:, None] + jnp.concatenate([b2_ref[...][None, :]] * _B, axis=1)
        for b in range(_B):
            out_ref[b] = r[:, b * _NOUT:(b + 1) * _NOUT]


def _edge_tile_map(i):
    t = jnp.clip(i - 1, 0, _NT - 1)
    return (t, 0)


def _dense_pipeline(a_flat, xv, m_in, m_out, W1, b1, Wm, bm, W2, b2):
    a_un = a_flat.reshape(_N, _N)
    f32 = jnp.float32
    zero2 = lambda i: (0, 0)
    out = pl.pallas_call(
        _fused_body,
        grid=(_GRID,),
        in_specs=[
            pl.BlockSpec((_N, _N), zero2),
            pl.BlockSpec((_B, _N, _NIN), lambda i: (0, 0, 0)),
            pl.BlockSpec((_TE, _N), _edge_tile_map),
            pl.BlockSpec((_TE, _N), _edge_tile_map),
            pl.BlockSpec((_NHID, _NIN), zero2),
            pl.BlockSpec((_NHID,), lambda i: (0,)),
            pl.BlockSpec((_NHID, 2 * _NHID), zero2),
            pl.BlockSpec((_NHID,), lambda i: (0,)),
            pl.BlockSpec((_NOUT, _NHID), zero2),
            pl.BlockSpec((_NOUT,), lambda i: (0,)),
        ],
        out_specs=pl.BlockSpec((_B, _N, _NOUT), lambda i: (0, 0, 0)),
        out_shape=jax.ShapeDtypeStruct((_B, _N, _NOUT), f32),
        scratch_shapes=[
            pltpu.VMEM((_N, _B * _NHID), jnp.bfloat16),
            pltpu.VMEM((_N, 1), f32),
            pltpu.VMEM((_N, _B * _NHID), f32),
        ],
        compiler_params=pltpu.CompilerParams(
            dimension_semantics=("arbitrary",)),
    )(a_un, xv, m_in, m_out, W1, b1, Wm, bm, W2, b2)
    return out


def kernel(x, edge_index, m_in, m_out, W1, b1, Wm, bm, W2, b2):
    xv = x.reshape(_B, _N, _NIN)
    a_flat = _build_adjacency(edge_index)
    return _dense_pipeline(a_flat, xv, m_in, m_out, W1, b1, Wm, bm, W2, b2)


# speedup vs baseline: 1.0137x; 1.0137x over previous
import functools

import jax
import jax.numpy as jnp
from jax import lax
from jax.experimental import pallas as pl
from jax.experimental.pallas import tpu as pltpu
from jax.experimental.pallas import tpu_sc as plsc

_B, _N, _E, _NIN, _NHID, _NOUT = 4, 1024, 16384, 128, 128, 128
_NW = 32
_ROWS_PER_W = _N // _NW
_BLK = _ROWS_PER_W * _N
_TE = 2048
_LANES = 16


def _build_adjacency(edge_index):
    mesh = plsc.VectorSubcoreMesh(core_axis_name="c", subcore_axis_name="s")

    @functools.partial(
        pl.kernel,
        out_type=jax.ShapeDtypeStruct((_N * _N,), jnp.float32),
        mesh=mesh,
        compiler_params=pltpu.CompilerParams(needs_layout_passes=False),
        scratch_types=[
            pltpu.VMEM((_E,), jnp.int32),
            pltpu.VMEM((_E,), jnp.int32),
            pltpu.VMEM((_BLK,), jnp.float32),
        ],
    )
    def scatter_kernel(edge_hbm, a_hbm, src_v, dst_v, blk_v):
        wid = lax.axis_index("s") * 2 + lax.axis_index("c")
        lo = wid * _ROWS_PER_W
        pltpu.sync_copy(edge_hbm.at[0], src_v)
        pltpu.sync_copy(edge_hbm.at[1], dst_v)

        @plsc.parallel_loop(0, _BLK // _LANES, unroll=16)
        def zero_body(i):
            blk_v[pl.ds(i * _LANES, _LANES)] = jnp.zeros((_LANES,), jnp.float32)

        ones = jnp.ones((_LANES,), jnp.float32)

        @plsc.parallel_loop(0, _E // _LANES, unroll=8)
        def scat_body(i):
            s = src_v[pl.ds(i * _LANES, _LANES)]
            d = dst_v[pl.ds(i * _LANES, _LANES)]
            rel = d - lo
            msk = (rel >= 0) & (rel < _ROWS_PER_W)
            idx = jnp.where(msk, rel * _N + s, 0)
            plsc.addupdate_scatter(blk_v, [idx], ones, mask=msk)
        pltpu.sync_copy(blk_v, a_hbm.at[pl.ds(lo * _N, _BLK)])

    return scatter_kernel(edge_index)


_NT = _E // _TE
_GRID = _NT + 2


def _fused_body(a_ref, x_ref, min_ref, mout_ref, w1_ref, b1_ref, wm_ref,
                bm_ref, w2_ref, b2_ref, out_ref, h_ref, dinv_ref, acc_ref):
    i = pl.program_id(0)

    @pl.when(i == 0)
    def _stage1():
        a = a_ref[...]
        deg = jnp.sum(a, axis=1) + 1.0
        dinv = lax.rsqrt(deg)
        w1 = w1_ref[...]
        p = jnp.concatenate(
            [lax.dot_general(x_ref[b], w1, (((1,), (1,)), ((), ())),
                             preferred_element_type=jnp.float32)
             for b in range(_B)], axis=1)
        q = p * dinv[:, None]
        agg = jnp.dot(a, q, preferred_element_type=jnp.float32) + q
        b1t = jnp.concatenate([b1_ref[...][None, :]] * _B, axis=1)
        h = jnp.maximum(agg * dinv[:, None] + b1t, 0.0)
        h_ref[...] = h.astype(jnp.bfloat16)
        dinv_ref[...] = dinv[:, None]

    @pl.when((i >= 1) & (i <= _NT))
    def _stage2():
        m_in_t = min_ref[...].astype(jnp.bfloat16)
        m_out_t = mout_ref[...].astype(jnp.bfloat16)
        h = h_ref[...]
        recv = jnp.dot(m_in_t, h, preferred_element_type=jnp.float32)
        send = jnp.dot(m_out_t, h, preferred_element_type=jnp.float32)
        wm = wm_ref[...]
        wma = wm[:, :_NHID]
        wmb = wm[:, _NHID:]
        bm = bm_ref[...][None, :]
        e_blocks = []
        for b in range(_B):
            rb = recv[:, b * _NHID:(b + 1) * _NHID]
            sb = send[:, b * _NHID:(b + 1) * _NHID]
            eb = (lax.dot_general(rb, wma, (((1,), (1,)), ((), ())),
                                  preferred_element_type=jnp.float32)
                  + lax.dot_general(sb, wmb, (((1,), (1,)), ((), ())),
                                    preferred_element_type=jnp.float32))
            e_blocks.append(jnp.maximum(eb + bm, 0.0))
        e_t = jnp.concatenate(e_blocks, axis=1).astype(jnp.bfloat16)
        contrib = lax.dot_general(m_in_t, e_t, (((0,), (0,)), ((), ())),
                                  preferred_element_type=jnp.float32)

        @pl.when(i == 1)
        def _():
            acc_ref[...] = contrib

        @pl.when(i > 1)
        def _():
            acc_ref[...] += contrib

    @pl.when(i == _NT + 1)
    def _stage3():
        dinv = dinv_ref[...][:, 0]
        node = acc_ref[...] * (1.0 / _N)
        w2 = w2_ref[...]
        q = jnp.concatenate(
            [lax.dot_general(node[:, b * _NHID:(b + 1) * _NHID], w2,
                             (((1,), (1,)), ((), ())),
                             preferred_element_type=jnp.float32)
             for b in range(_B)], axis=1)
        qq = q * dinv[:, None]
        r = jnp.dot(a_ref[...], qq, preferred_element_type=jnp.float32) + qq
        r = r * dinv[:, None] + jnp.concatenate([b2_ref[...][None, :]] * _B, axis=1)
        for b in range(_B):
            out_ref[b] = r[:, b * _NOUT:(b + 1) * _NOUT]


def _edge_tile_map(i):
    t = jnp.clip(i - 1, 0, _NT - 1)
    return (t, 0)


def _dense_pipeline(a_flat, xv, m_in, m_out, W1, b1, Wm, bm, W2, b2):
    a_un = a_flat.reshape(_N, _N)
    f32 = jnp.float32
    zero2 = lambda i: (0, 0)
    out = pl.pallas_call(
        _fused_body,
        grid=(_GRID,),
        in_specs=[
            pl.BlockSpec((_N, _N), zero2),
            pl.BlockSpec((_B, _N, _NIN), lambda i: (0, 0, 0)),
            pl.BlockSpec((_TE, _N), _edge_tile_map),
            pl.BlockSpec((_TE, _N), _edge_tile_map),
            pl.BlockSpec((_NHID, _NIN), zero2),
            pl.BlockSpec((_NHID,), lambda i: (0,)),
            pl.BlockSpec((_NHID, 2 * _NHID), zero2),
            pl.BlockSpec((_NHID,), lambda i: (0,)),
            pl.BlockSpec((_NOUT, _NHID), zero2),
            pl.BlockSpec((_NOUT,), lambda i: (0,)),
        ],
        out_specs=pl.BlockSpec((_B, _N, _NOUT), lambda i: (0, 0, 0)),
        out_shape=jax.ShapeDtypeStruct((_B, _N, _NOUT), f32),
        scratch_shapes=[
            pltpu.VMEM((_N, _B * _NHID), jnp.bfloat16),
            pltpu.VMEM((_N, 1), f32),
            pltpu.VMEM((_N, _B * _NHID), f32),
        ],
        compiler_params=pltpu.CompilerParams(
            dimension_semantics=("arbitrary",)),
    )(a_un, xv, m_in, m_out, W1, b1, Wm, bm, W2, b2)
    return out


def kernel(x, edge_index, m_in, m_out, W1, b1, Wm, bm, W2, b2):
    xv = x.reshape(_B, _N, _NIN)
    a_flat = _build_adjacency(edge_index)
    return _dense_pipeline(a_flat, xv, m_in, m_out, W1, b1, Wm, bm, W2, b2)


# speedup vs baseline: 1.1106x; 1.0956x over previous
import functools

import jax
import jax.numpy as jnp
from jax import lax
from jax.experimental import pallas as pl
from jax.experimental.pallas import tpu as pltpu
from jax.experimental.pallas import tpu_sc as plsc

_B, _N, _E, _NIN, _NHID, _NOUT = 4, 1024, 16384, 128, 128, 128
_NW = 32
_ROWS_PER_W = _N // _NW
_BLK = _ROWS_PER_W * _N
_TE = 2048
_LANES = 16


def _build_adjacency(edge_index):
    mesh = plsc.VectorSubcoreMesh(core_axis_name="c", subcore_axis_name="s")

    @functools.partial(
        pl.kernel,
        out_type=jax.ShapeDtypeStruct((_N * _N,), jnp.float32),
        mesh=mesh,
        compiler_params=pltpu.CompilerParams(needs_layout_passes=False),
        scratch_types=[
            pltpu.VMEM((_E,), jnp.int32),
            pltpu.VMEM((_E,), jnp.int32),
            pltpu.VMEM((_BLK,), jnp.float32),
            pltpu.SemaphoreType.DMA,
        ],
    )
    def scatter_kernel(edge_hbm, a_hbm, src_v, dst_v, blk_v, sem):
        wid = lax.axis_index("s") * 2 + lax.axis_index("c")
        lo = wid * _ROWS_PER_W
        cp_src = pltpu.async_copy(edge_hbm.at[0], src_v, sem)
        cp_dst = pltpu.async_copy(edge_hbm.at[1], dst_v, sem)

        @plsc.parallel_loop(0, _BLK // _LANES, unroll=16)
        def zero_body(i):
            blk_v[pl.ds(i * _LANES, _LANES)] = jnp.zeros((_LANES,), jnp.float32)

        cp_src.wait()
        cp_dst.wait()

        ones = jnp.ones((_LANES,), jnp.float32)

        @plsc.parallel_loop(0, _E // _LANES, unroll=8)
        def scat_body(i):
            s = src_v[pl.ds(i * _LANES, _LANES)]
            d = dst_v[pl.ds(i * _LANES, _LANES)]
            rel = d - lo
            msk = (rel >= 0) & (rel < _ROWS_PER_W)
            idx = jnp.where(msk, rel * _N + s, 0)
            plsc.addupdate_scatter(blk_v, [idx], ones, mask=msk)
        pltpu.sync_copy(blk_v, a_hbm.at[pl.ds(lo * _N, _BLK)])

    return scatter_kernel(edge_index)


_NT = _E // _TE
_GRID = _NT + 2


def _fused_body(a_ref, x_ref, min_ref, mout_ref, w1_ref, b1_ref, wm_ref,
                bm_ref, w2_ref, b2_ref, out_ref, ha_ref, hb_ref, dinv_ref,
                acc_ref):
    i = pl.program_id(0)

    @pl.when(i == 0)
    def _stage1():
        a = a_ref[...]
        deg = jnp.sum(a, axis=1) + 1.0
        dinv = lax.rsqrt(deg)
        w1 = w1_ref[...]
        p = jnp.concatenate(
            [lax.dot_general(x_ref[b], w1, (((1,), (1,)), ((), ())),
                             preferred_element_type=jnp.float32)
             for b in range(_B)], axis=1)
        q = p * dinv[:, None]
        agg = jnp.dot(a, q, preferred_element_type=jnp.float32) + q
        b1t = jnp.concatenate([b1_ref[...][None, :]] * _B, axis=1)
        h = jnp.maximum(agg * dinv[:, None] + b1t, 0.0)
        wm = wm_ref[...]
        wma = wm[:, :_NHID]
        wmb = wm[:, _NHID:]
        ha_blocks, hb_blocks = [], []
        for b in range(_B):
            hbk = h[:, b * _NHID:(b + 1) * _NHID]
            ha_blocks.append(
                lax.dot_general(hbk, wma, (((1,), (1,)), ((), ())),
                                preferred_element_type=jnp.float32))
            hb_blocks.append(
                lax.dot_general(hbk, wmb, (((1,), (1,)), ((), ())),
                                preferred_element_type=jnp.float32))
        ha_ref[...] = jnp.concatenate(ha_blocks, axis=1).astype(jnp.bfloat16)
        hb_ref[...] = jnp.concatenate(hb_blocks, axis=1).astype(jnp.bfloat16)
        dinv_ref[...] = dinv[:, None]

    @pl.when((i >= 1) & (i <= _NT))
    def _stage2():
        m_in_t = min_ref[...].astype(jnp.bfloat16)
        m_out_t = mout_ref[...].astype(jnp.bfloat16)
        pre = (jnp.dot(m_in_t, ha_ref[...], preferred_element_type=jnp.float32)
               + jnp.dot(m_out_t, hb_ref[...],
                         preferred_element_type=jnp.float32))
        bmt = jnp.concatenate([bm_ref[...][None, :]] * _B, axis=1)
        e_t = jnp.maximum(pre + bmt, 0.0).astype(jnp.bfloat16)
        contrib = lax.dot_general(m_in_t, e_t, (((0,), (0,)), ((), ())),
                                  preferred_element_type=jnp.float32)

        @pl.when(i == 1)
        def _():
            acc_ref[...] = contrib

        @pl.when(i > 1)
        def _():
            acc_ref[...] += contrib

    @pl.when(i == _NT + 1)
    def _stage3():
        dinv = dinv_ref[...][:, 0]
        node = acc_ref[...] * (1.0 / _N)
        w2 = w2_ref[...]
        q = jnp.concatenate(
            [lax.dot_general(node[:, b * _NHID:(b + 1) * _NHID], w2,
                             (((1,), (1,)), ((), ())),
                             preferred_element_type=jnp.float32)
             for b in range(_B)], axis=1)
        qq = q * dinv[:, None]
        r = jnp.dot(a_ref[...], qq, preferred_element_type=jnp.float32) + qq
        r = r * dinv[:, None] + jnp.concatenate([b2_ref[...][None, :]] * _B, axis=1)
        for b in range(_B):
            out_ref[b] = r[:, b * _NOUT:(b + 1) * _NOUT]


def _edge_tile_map(i):
    t = jnp.clip(i - 1, 0, _NT - 1)
    return (t, 0)


def _dense_pipeline(a_flat, xv, m_in, m_out, W1, b1, Wm, bm, W2, b2):
    a_un = a_flat.reshape(_N, _N)
    f32 = jnp.float32
    zero2 = lambda i: (0, 0)
    out = pl.pallas_call(
        _fused_body,
        grid=(_GRID,),
        in_specs=[
            pl.BlockSpec((_N, _N), zero2),
            pl.BlockSpec((_B, _N, _NIN), lambda i: (0, 0, 0)),
            pl.BlockSpec((_TE, _N), _edge_tile_map),
            pl.BlockSpec((_TE, _N), _edge_tile_map),
            pl.BlockSpec((_NHID, _NIN), zero2),
            pl.BlockSpec((_NHID,), lambda i: (0,)),
            pl.BlockSpec((_NHID, 2 * _NHID), zero2),
            pl.BlockSpec((_NHID,), lambda i: (0,)),
            pl.BlockSpec((_NOUT, _NHID), zero2),
            pl.BlockSpec((_NOUT,), lambda i: (0,)),
        ],
        out_specs=pl.BlockSpec((_B, _N, _NOUT), lambda i: (0, 0, 0)),
        out_shape=jax.ShapeDtypeStruct((_B, _N, _NOUT), f32),
        scratch_shapes=[
            pltpu.VMEM((_N, _B * _NHID), jnp.bfloat16),
            pltpu.VMEM((_N, _B * _NHID), jnp.bfloat16),
            pltpu.VMEM((_N, 1), f32),
            pltpu.VMEM((_N, _B * _NHID), f32),
        ],
        compiler_params=pltpu.CompilerParams(
            dimension_semantics=("arbitrary",)),
    )(a_un, xv, m_in, m_out, W1, b1, Wm, bm, W2, b2)
    return out


def kernel(x, edge_index, m_in, m_out, W1, b1, Wm, bm, W2, b2):
    xv = x.reshape(_B, _N, _NIN)
    a_flat = _build_adjacency(edge_index)
    return _dense_pipeline(a_flat, xv, m_in, m_out, W1, b1, Wm, bm, W2, b2)
